# baseline (speedup 1.0000x reference)
import functools

import jax
import jax.numpy as jnp
from jax import lax
from jax.experimental import pallas as pl
from jax.experimental.pallas import tpu as pltpu
from jax.experimental.pallas import tpu_sc as plsc

NUM_FEATURES = 2344
ACC = 8
PAIRS = ACC // 2
BATCH = 16384
L = 50

NC = 2
NS = 16
NW = NC * NS
ROWS_PER_W = BATCH // NW
CHUNK = 256
NCHUNK = ROWS_PER_W // CHUNK
CGROUPS = CHUNK // 16
TBL_WORDS = NUM_FEATURES * PAIRS


def _sc_body(bf_hbm, wf_hbm, stm_hbm, tbl_hbm, cst_hbm, out_hbm,
             bf_v, wf_v, stm_v, tbl_v, cst_v, out_v):
    wid = lax.axis_index("s") * NC + lax.axis_index("c")
    base = wid * ROWS_PER_W

    pltpu.sync_copy(stm_hbm.at[pl.ds(base, ROWS_PER_W)], stm_v)
    pltpu.sync_copy(tbl_hbm, tbl_v)
    pltpu.sync_copy(cst_hbm, cst_v)

    iota16 = lax.iota(jnp.int32, 16)
    bias = [cst_v[pl.ds(c * 16, 16)] for c in range(ACC)]
    wfir = [cst_v[pl.ds((ACC + c) * 16, 16)] for c in range(ACC)]
    wsec = [cst_v[pl.ds((2 * ACC + c) * 16, 16)] for c in range(ACC)]
    l2b = cst_v[pl.ds(3 * ACC * 16, 16)]

    zero = jnp.zeros((16,), jnp.float32)

    def unpacked(word):
        pair = plsc.bitcast(word, jnp.bfloat16)
        return plsc.unpack(pair, format=plsc.PackFormat.INTERLEAVED,
                           preferred_element_type=jnp.float32)

    for h in range(NCHUNK):
        row0 = base + h * CHUNK
        pltpu.sync_copy(bf_hbm.at[pl.ds(row0, CHUNK)], bf_v)
        pltpu.sync_copy(wf_hbm.at[pl.ds(row0, CHUNK)], wf_v)

        def group_body(g, carry):
            rows = g * 16 + iota16

            def l_body(l, accs):
                accb, accw = accs[:ACC], accs[ACC:]
                lvec = jnp.full((16,), l, dtype=jnp.int32)
                tb = plsc.load_gather(bf_v, [rows, lvec]) * PAIRS
                tw = plsc.load_gather(wf_v, [rows, lvec]) * PAIRS
                naccb, naccw = [], []
                for p in range(PAIRS):
                    eb, ob = unpacked(plsc.load_gather(tbl_v, [tb + p]))
                    ew, ow = unpacked(plsc.load_gather(tbl_v, [tw + p]))
                    naccb += [accb[2 * p] + eb, accb[2 * p + 1] + ob]
                    naccw += [accw[2 * p] + ew, accw[2 * p + 1] + ow]
                return tuple(naccb) + tuple(naccw)

            accs = lax.fori_loop(0, L, l_body, (zero,) * (2 * ACC))
            accb, accw = accs[:ACC], accs[ACC:]

            out_rows = h * CHUNK + g * 16 + iota16
            stm_g = plsc.load_gather(stm_v, [out_rows])
            m = stm_g == 0
            o = l2b
            for c in range(ACC):
                cb = jnp.where(m, wfir[c], wsec[c])
                cw = jnp.where(m, wsec[c], wfir[c])
                ab = jnp.clip(accb[c] + bias[c], 0.0, 1.0)
                aw = jnp.clip(accw[c] + bias[c], 0.0, 1.0)
                o = o + cb * ab + cw * aw
            plsc.store_scatter(out_v, [out_rows], o)
            return carry

        lax.fori_loop(0, CGROUPS, group_body, 0)

    pltpu.sync_copy(out_v, out_hbm.at[pl.ds(base, ROWS_PER_W)])


@functools.lru_cache(maxsize=1)
def _sc_kernel():
    mesh = plsc.VectorSubcoreMesh(core_axis_name="c", subcore_axis_name="s",
                                  num_cores=NC, num_subcores=NS)
    return pl.kernel(
        _sc_body,
        out_type=jax.ShapeDtypeStruct((BATCH,), jnp.float32),
        mesh=mesh,
        compiler_params=pltpu.CompilerParams(needs_layout_passes=False,
                                             use_tc_tiling_on_sc=True),
        scratch_types=[
            pltpu.VMEM((CHUNK, L), jnp.int32),
            pltpu.VMEM((CHUNK, L), jnp.int32),
            pltpu.VMEM((ROWS_PER_W,), jnp.int32),
            pltpu.VMEM((TBL_WORDS,), jnp.int32),
            pltpu.VMEM((26 * 16,), jnp.float32),
            pltpu.VMEM((ROWS_PER_W,), jnp.float32),
        ],
    )


def kernel(black_features, white_features, stm, l1_weight, l1_bias,
           l2_weight, l2_bias):
    bf = black_features.astype(jnp.int32)
    wf = white_features.astype(jnp.int32)
    stm32 = stm.astype(jnp.int32)
    tbl = lax.bitcast_convert_type(
        l1_weight.astype(jnp.bfloat16).reshape(NUM_FEATURES, PAIRS, 2),
        jnp.int32).reshape(-1)

    w = l2_weight.reshape(2 * ACC)
    cst = jnp.concatenate([
        jnp.broadcast_to(l1_bias[:, None], (ACC, 16)),
        jnp.broadcast_to(w[:ACC, None], (ACC, 16)),
        jnp.broadcast_to(w[ACC:, None], (ACC, 16)),
        jnp.broadcast_to(l2_bias.reshape(1, 1), (1, 16)),
        jnp.zeros((1, 16), jnp.float32),
    ], axis=0).reshape(-1)

    out = _sc_kernel()(bf, wf, stm32, tbl, cst)
    return out.reshape(BATCH, 1)


# speedup vs baseline: 1.0989x; 1.0989x over previous
import functools

import jax
import jax.numpy as jnp
from jax import lax
from jax.experimental import pallas as pl
from jax.experimental.pallas import tpu as pltpu
from jax.experimental.pallas import tpu_sc as plsc

NUM_FEATURES = 2344
ACC = 8
PAIRS = ACC // 2
BATCH = 16384
L = 50

NC = 2
NS = 16
NW = NC * NS
ROWS_PER_W = BATCH // NW
GROUPS = ROWS_PER_W // 16
IDX_PER_W = ROWS_PER_W * L
TBL_STRIDE = 5
TBL_WORDS = NUM_FEATURES * TBL_STRIDE


def _sc_body(bf_hbm, wf_hbm, stm_hbm, tbl_hbm, cst_hbm, out_hbm,
             bf_v, wf_v, stm_v, tbl_v, cst_v, out_v):
    wid = lax.axis_index("s") * NC + lax.axis_index("c")
    base = wid * ROWS_PER_W

    pltpu.sync_copy(bf_hbm.at[pl.ds(base * L, IDX_PER_W)], bf_v)
    pltpu.sync_copy(wf_hbm.at[pl.ds(base * L, IDX_PER_W)], wf_v)
    pltpu.sync_copy(stm_hbm.at[pl.ds(base, ROWS_PER_W)], stm_v)
    pltpu.sync_copy(tbl_hbm, tbl_v)
    pltpu.sync_copy(cst_hbm, cst_v)

    iota16 = lax.iota(jnp.int32, 16)
    bias = [cst_v[c] for c in range(ACC)]
    wfir = [cst_v[ACC + c] for c in range(ACC)]
    wsec = [cst_v[2 * ACC + c] for c in range(ACC)]
    l2b = cst_v[3 * ACC]

    zero = jnp.zeros((16,), jnp.float32)

    def unpacked(word):
        pair = plsc.bitcast(word, jnp.bfloat16)
        return plsc.unpack(pair, format=plsc.PackFormat.INTERLEAVED,
                           preferred_element_type=jnp.float32)

    def group_body(g, carry):
        rows = g * 16 + iota16
        rowbase = rows * L

        def l_body(l, accs):
            accb, accw = accs[:ACC], accs[ACC:]
            tb = plsc.load_gather(bf_v, [rowbase + l]) * TBL_STRIDE
            tw = plsc.load_gather(wf_v, [rowbase + l]) * TBL_STRIDE
            naccb, naccw = [], []
            for p in range(PAIRS):
                eb, ob = unpacked(plsc.load_gather(tbl_v, [tb + p]))
                ew, ow = unpacked(plsc.load_gather(tbl_v, [tw + p]))
                naccb += [accb[2 * p] + eb, accb[2 * p + 1] + ob]
                naccw += [accw[2 * p] + ew, accw[2 * p + 1] + ow]
            return tuple(naccb) + tuple(naccw)

        accs = lax.fori_loop(0, L, l_body, (zero,) * (2 * ACC))
        accb, accw = accs[:ACC], accs[ACC:]

        stm_g = plsc.load_gather(stm_v, [rows])
        m = stm_g == 0
        o = l2b
        for c in range(ACC):
            cb = jnp.where(m, wfir[c], wsec[c])
            cw = jnp.where(m, wsec[c], wfir[c])
            ab = jnp.clip(accb[c] + bias[c], 0.0, 1.0)
            aw = jnp.clip(accw[c] + bias[c], 0.0, 1.0)
            o = o + cb * ab + cw * aw
        plsc.store_scatter(out_v, [rows], o)
        return carry

    lax.fori_loop(0, GROUPS, group_body, 0)
    pltpu.sync_copy(out_v, out_hbm.at[pl.ds(base, ROWS_PER_W)])


@functools.lru_cache(maxsize=1)
def _sc_kernel():
    mesh = plsc.VectorSubcoreMesh(core_axis_name="c", subcore_axis_name="s",
                                  num_cores=NC, num_subcores=NS)
    return pl.kernel(
        _sc_body,
        out_type=jax.ShapeDtypeStruct((BATCH,), jnp.float32),
        mesh=mesh,
        compiler_params=pltpu.CompilerParams(needs_layout_passes=False),
        scratch_types=[
            pltpu.VMEM((IDX_PER_W,), jnp.int32),
            pltpu.VMEM((IDX_PER_W,), jnp.int32),
            pltpu.VMEM((ROWS_PER_W,), jnp.int32),
            pltpu.VMEM((TBL_WORDS,), jnp.int32),
            pltpu.VMEM((32, 16), jnp.float32),
            pltpu.VMEM((ROWS_PER_W,), jnp.float32),
        ],
    )


def kernel(black_features, white_features, stm, l1_weight, l1_bias,
           l2_weight, l2_bias):
    bf = black_features.astype(jnp.int32).reshape(-1)
    wf = white_features.astype(jnp.int32).reshape(-1)
    stm32 = stm.astype(jnp.int32)
    packed = lax.bitcast_convert_type(
        l1_weight.astype(jnp.bfloat16).reshape(NUM_FEATURES, PAIRS, 2),
        jnp.int32)
    tbl = jnp.pad(packed, ((0, 0), (0, TBL_STRIDE - PAIRS))).reshape(-1)

    w = l2_weight.reshape(2 * ACC)
    cst = jnp.concatenate([
        jnp.broadcast_to(l1_bias[:, None], (ACC, 16)),
        jnp.broadcast_to(w[:ACC, None], (ACC, 16)),
        jnp.broadcast_to(w[ACC:, None], (ACC, 16)),
        jnp.broadcast_to(l2_bias.reshape(1, 1), (1, 16)),
        jnp.zeros((7, 16), jnp.float32),
    ], axis=0)

    out = _sc_kernel()(bf, wf, stm32, tbl, cst)
    return out.reshape(BATCH, 1)


# speedup vs baseline: 1.1910x; 1.0838x over previous
import functools

import jax
import jax.numpy as jnp
from jax import lax
from jax.experimental import pallas as pl
from jax.experimental.pallas import tpu as pltpu
from jax.experimental.pallas import tpu_sc as plsc

NUM_FEATURES = 2344
ACC = 8
PAIRS = ACC // 2
BATCH = 16384
L = 50

NC = 2
NS = 16
NW = NC * NS
ROWS_PER_W = BATCH // NW
CHUNK = 256
NCHUNK = ROWS_PER_W // CHUNK
CGROUPS = CHUNK // 16
TBL_STRIDE = 5
TBL_WORDS = NUM_FEATURES * TBL_STRIDE
TSTRIDE = 17
T_WORDS = CGROUPS * L * TSTRIDE


def _sc_body(bf_hbm, wf_hbm, stm_hbm, tbl_hbm, cst_hbm, out_hbm,
             raw_b, raw_w, tb_v, tw_v, stm_v, tbl_v, cst_v, out_v):
    wid = lax.axis_index("s") * NC + lax.axis_index("c")
    base = wid * ROWS_PER_W

    pltpu.sync_copy(stm_hbm.at[pl.ds(base, ROWS_PER_W)], stm_v)
    pltpu.sync_copy(tbl_hbm, tbl_v)
    pltpu.sync_copy(cst_hbm, cst_v)

    iota16 = lax.iota(jnp.int32, 16)
    bias = [cst_v[pl.ds(c * 16, 16)] for c in range(ACC)]
    wfir = [cst_v[pl.ds((ACC + c) * 16, 16)] for c in range(ACC)]
    wsec = [cst_v[pl.ds((2 * ACC + c) * 16, 16)] for c in range(ACC)]
    l2b = cst_v[pl.ds(3 * ACC * 16, 16)]

    zero = jnp.zeros((16,), jnp.float32)
    tstep = iota16 * TSTRIDE

    def unpacked(word):
        pair = plsc.bitcast(word, jnp.bfloat16)
        return plsc.unpack(pair, format=plsc.PackFormat.INTERLEAVED,
                           preferred_element_type=jnp.float32)

    for h in range(NCHUNK):
        row0 = base + h * CHUNK
        pltpu.sync_copy(bf_hbm.at[pl.ds(row0, CHUNK)], raw_b)
        pltpu.sync_copy(wf_hbm.at[pl.ds(row0, CHUNK)], raw_w)

        def compact_body(r, carry):
            g = r >> 4
            j = r & 15
            gl = g * L
            for l0 in (0, 16, 32, L - 16):
                widx = (gl + l0) * TSTRIDE + j + tstep
                vb = raw_b.at[r][pl.ds(l0, 16)] * TBL_STRIDE
                vw = raw_w.at[r][pl.ds(l0, 16)] * TBL_STRIDE
                plsc.store_scatter(tb_v, [widx], vb)
                plsc.store_scatter(tw_v, [widx], vw)
            return carry

        lax.fori_loop(0, CHUNK, compact_body, 0)

        def group_body(g, carry):
            gl = g * L

            def l_body(l, accs):
                accb, accw = accs[:ACC], accs[ACC:]
                tbase = (gl + l) * TSTRIDE
                tb = tb_v[pl.ds(tbase, 16)]
                tw = tw_v[pl.ds(tbase, 16)]
                naccb, naccw = [], []
                for p in range(PAIRS):
                    eb, ob = unpacked(plsc.load_gather(tbl_v, [tb + p]))
                    ew, ow = unpacked(plsc.load_gather(tbl_v, [tw + p]))
                    naccb += [accb[2 * p] + eb, accb[2 * p + 1] + ob]
                    naccw += [accw[2 * p] + ew, accw[2 * p + 1] + ow]
                return tuple(naccb) + tuple(naccw)

            accs = lax.fori_loop(0, L, l_body, (zero,) * (2 * ACC))
            accb, accw = accs[:ACC], accs[ACC:]

            out_rows = h * CHUNK + g * 16 + iota16
            stm_g = plsc.load_gather(stm_v, [out_rows])
            m = stm_g == 0
            o = l2b
            for c in range(ACC):
                cb = jnp.where(m, wfir[c], wsec[c])
                cw = jnp.where(m, wsec[c], wfir[c])
                ab = jnp.clip(accb[c] + bias[c], 0.0, 1.0)
                aw = jnp.clip(accw[c] + bias[c], 0.0, 1.0)
                o = o + cb * ab + cw * aw
            plsc.store_scatter(out_v, [out_rows], o)
            return carry

        lax.fori_loop(0, CGROUPS, group_body, 0)

    pltpu.sync_copy(out_v, out_hbm.at[pl.ds(base, ROWS_PER_W)])


@functools.lru_cache(maxsize=1)
def _sc_kernel():
    mesh = plsc.VectorSubcoreMesh(core_axis_name="c", subcore_axis_name="s",
                                  num_cores=NC, num_subcores=NS)
    return pl.kernel(
        _sc_body,
        out_type=jax.ShapeDtypeStruct((BATCH,), jnp.float32),
        mesh=mesh,
        compiler_params=pltpu.CompilerParams(needs_layout_passes=False,
                                             use_tc_tiling_on_sc=True),
        scratch_types=[
            pltpu.VMEM((CHUNK, L), jnp.int32),
            pltpu.VMEM((CHUNK, L), jnp.int32),
            pltpu.VMEM((T_WORDS,), jnp.int32),
            pltpu.VMEM((T_WORDS,), jnp.int32),
            pltpu.VMEM((ROWS_PER_W,), jnp.int32),
            pltpu.VMEM((TBL_WORDS,), jnp.int32),
            pltpu.VMEM((26 * 16,), jnp.float32),
            pltpu.VMEM((ROWS_PER_W,), jnp.float32),
        ],
    )


def kernel(black_features, white_features, stm, l1_weight, l1_bias,
           l2_weight, l2_bias):
    bf = black_features.astype(jnp.int32)
    wf = white_features.astype(jnp.int32)
    stm32 = stm.astype(jnp.int32)
    packed = lax.bitcast_convert_type(
        l1_weight.astype(jnp.bfloat16).reshape(NUM_FEATURES, PAIRS, 2),
        jnp.int32)
    tbl = jnp.pad(packed, ((0, 0), (0, TBL_STRIDE - PAIRS))).reshape(-1)

    w = l2_weight.reshape(2 * ACC)
    cst = jnp.concatenate([
        jnp.broadcast_to(l1_bias[:, None], (ACC, 16)),
        jnp.broadcast_to(w[:ACC, None], (ACC, 16)),
        jnp.broadcast_to(w[ACC:, None], (ACC, 16)),
        jnp.broadcast_to(l2_bias.reshape(1, 1), (1, 16)),
        jnp.zeros((1, 16), jnp.float32),
    ], axis=0).reshape(-1)

    out = _sc_kernel()(bf, wf, stm32, tbl, cst)
    return out.reshape(BATCH, 1)


# speedup vs baseline: 1.1945x; 1.0029x over previous
import functools

import jax
import jax.numpy as jnp
from jax import lax
from jax.experimental import pallas as pl
from jax.experimental.pallas import tpu as pltpu
from jax.experimental.pallas import tpu_sc as plsc

NUM_FEATURES = 2344
ACC = 8
PAIRS = ACC // 2
BATCH = 16384
L = 50

NC = 2
NS = 16
NW = NC * NS
ROWS_PER_W = BATCH // NW
CHUNK = 256
NCHUNK = ROWS_PER_W // CHUNK
CGROUPS = CHUNK // 16
TBL_STRIDE = 5
TBL_WORDS = NUM_FEATURES * TBL_STRIDE
TSTRIDE = 17
T_WORDS = CGROUPS * L * TSTRIDE


def _sc_body(bf_hbm, wf_hbm, stm_hbm, tbl_hbm, cst_hbm, out_hbm,
             raw_b, raw_w, tb_v, tw_v, stm_v, tbl_v, cst_v, out_v):
    wid = lax.axis_index("s") * NC + lax.axis_index("c")
    base = wid * ROWS_PER_W

    pltpu.sync_copy(stm_hbm.at[pl.ds(base, ROWS_PER_W)], stm_v)
    pltpu.sync_copy(tbl_hbm, tbl_v)
    pltpu.sync_copy(cst_hbm, cst_v)

    iota16 = lax.iota(jnp.int32, 16)
    bias = [cst_v[pl.ds(c * 16, 16)] for c in range(ACC)]
    wfir = [cst_v[pl.ds((ACC + c) * 16, 16)] for c in range(ACC)]
    wsec = [cst_v[pl.ds((2 * ACC + c) * 16, 16)] for c in range(ACC)]
    l2b = cst_v[pl.ds(3 * ACC * 16, 16)]

    zero = jnp.zeros((16,), jnp.float32)
    tstep = iota16 * TSTRIDE

    def unpacked(word):
        pair = plsc.bitcast(word, jnp.bfloat16)
        return plsc.unpack(pair, format=plsc.PackFormat.INTERLEAVED,
                           preferred_element_type=jnp.float32)

    def chunk_body(h, carry0):
        row0 = base + h * CHUNK
        pltpu.sync_copy(bf_hbm.at[pl.ds(row0, CHUNK)], raw_b)
        pltpu.sync_copy(wf_hbm.at[pl.ds(row0, CHUNK)], raw_w)

        def compact_body(r, carry):
            g = r >> 4
            j = r & 15
            gl = g * L
            for l0 in (0, 16, 32, L - 16):
                widx = (gl + l0) * TSTRIDE + j + tstep
                vb = raw_b.at[r][pl.ds(l0, 16)] * TBL_STRIDE
                vw = raw_w.at[r][pl.ds(l0, 16)] * TBL_STRIDE
                plsc.store_scatter(tb_v, [widx], vb)
                plsc.store_scatter(tw_v, [widx], vw)
            return carry

        lax.fori_loop(0, CHUNK, compact_body, 0)

        def group_body(g, carry):
            gl = g * L

            def l_body(l, accs):
                accb, accw = accs[:ACC], accs[ACC:]
                tbase = (gl + l) * TSTRIDE
                tb = tb_v[pl.ds(tbase, 16)]
                tw = tw_v[pl.ds(tbase, 16)]
                naccb, naccw = [], []
                for p in range(PAIRS):
                    eb, ob = unpacked(plsc.load_gather(tbl_v, [tb + p]))
                    ew, ow = unpacked(plsc.load_gather(tbl_v, [tw + p]))
                    naccb += [accb[2 * p] + eb, accb[2 * p + 1] + ob]
                    naccw += [accw[2 * p] + ew, accw[2 * p + 1] + ow]
                return tuple(naccb) + tuple(naccw)

            accs = lax.fori_loop(0, L, l_body, (zero,) * (2 * ACC))
            accb, accw = accs[:ACC], accs[ACC:]

            out_rows = h * CHUNK + g * 16 + iota16
            stm_g = plsc.load_gather(stm_v, [out_rows])
            m = stm_g == 0
            o = l2b
            for c in range(ACC):
                cb = jnp.where(m, wfir[c], wsec[c])
                cw = jnp.where(m, wsec[c], wfir[c])
                ab = jnp.clip(accb[c] + bias[c], 0.0, 1.0)
                aw = jnp.clip(accw[c] + bias[c], 0.0, 1.0)
                o = o + cb * ab + cw * aw
            plsc.store_scatter(out_v, [out_rows], o)
            return carry

        lax.fori_loop(0, CGROUPS, group_body, 0)
        return carry0

    lax.fori_loop(0, NCHUNK, chunk_body, 0)
    pltpu.sync_copy(out_v, out_hbm.at[pl.ds(base, ROWS_PER_W)])


@functools.lru_cache(maxsize=1)
def _sc_kernel():
    mesh = plsc.VectorSubcoreMesh(core_axis_name="c", subcore_axis_name="s",
                                  num_cores=NC, num_subcores=NS)
    return pl.kernel(
        _sc_body,
        out_type=jax.ShapeDtypeStruct((BATCH,), jnp.float32),
        mesh=mesh,
        compiler_params=pltpu.CompilerParams(needs_layout_passes=False,
                                             use_tc_tiling_on_sc=True),
        scratch_types=[
            pltpu.VMEM((CHUNK, L), jnp.int32),
            pltpu.VMEM((CHUNK, L), jnp.int32),
            pltpu.VMEM((T_WORDS,), jnp.int32),
            pltpu.VMEM((T_WORDS,), jnp.int32),
            pltpu.VMEM((ROWS_PER_W,), jnp.int32),
            pltpu.VMEM((TBL_WORDS,), jnp.int32),
            pltpu.VMEM((26 * 16,), jnp.float32),
            pltpu.VMEM((ROWS_PER_W,), jnp.float32),
        ],
    )


def kernel(black_features, white_features, stm, l1_weight, l1_bias,
           l2_weight, l2_bias):
    bf = black_features.astype(jnp.int32)
    wf = white_features.astype(jnp.int32)
    stm32 = stm.astype(jnp.int32)
    packed = lax.bitcast_convert_type(
        l1_weight.astype(jnp.bfloat16).reshape(NUM_FEATURES, PAIRS, 2),
        jnp.int32)
    tbl = jnp.pad(packed, ((0, 0), (0, TBL_STRIDE - PAIRS))).reshape(-1)

    w = l2_weight.reshape(2 * ACC)
    cst = jnp.concatenate([
        jnp.broadcast_to(l1_bias[:, None], (ACC, 16)),
        jnp.broadcast_to(w[:ACC, None], (ACC, 16)),
        jnp.broadcast_to(w[ACC:, None], (ACC, 16)),
        jnp.broadcast_to(l2_bias.reshape(1, 1), (1, 16)),
        jnp.zeros((1, 16), jnp.float32),
    ], axis=0).reshape(-1)

    out = _sc_kernel()(bf, wf, stm32, tbl, cst)
    return out.reshape(BATCH, 1)


# speedup vs baseline: 1.3014x; 1.0895x over previous
import functools

import jax
import jax.numpy as jnp
from jax import lax
from jax.experimental import pallas as pl
from jax.experimental.pallas import tpu as pltpu
from jax.experimental.pallas import tpu_sc as plsc

NUM_FEATURES = 2344
ACC = 8
PAIRS = ACC // 2
BATCH = 16384
L = 50

NC = 2
NS = 16
NW = NC * NS
ROWS_PER_W = BATCH // NW
CHUNK = 128
NCHUNK = ROWS_PER_W // CHUNK
CGROUPS = CHUNK // 16
TBL_STRIDE = 5
TBL_WORDS = NUM_FEATURES * TBL_STRIDE
TSTRIDE = 17
T_WORDS = CGROUPS * L * TSTRIDE


def _sc_body(bf_hbm, wf_hbm, stm_hbm, tbl_hbm, cst_hbm, out_hbm,
             raw_b0, raw_w0, raw_b1, raw_w1, tb_v, tw_v,
             stm_v, tbl_v, cst_v, out_v, sb0, sw0, sb1, sw1):
    wid = lax.axis_index("s") * NC + lax.axis_index("c")
    base = wid * ROWS_PER_W

    raws = [(raw_b0, raw_w0, sb0, sw0), (raw_b1, raw_w1, sb1, sw1)]

    def issue(h):
        rb, rw, sb, sw = raws[h % 2]
        row0 = base + h * CHUNK
        db = pltpu.async_copy(bf_hbm.at[pl.ds(row0, CHUNK)], rb, sb)
        dw = pltpu.async_copy(wf_hbm.at[pl.ds(row0, CHUNK)], rw, sw)
        return db, dw

    pending = issue(0)

    pltpu.sync_copy(stm_hbm.at[pl.ds(base, ROWS_PER_W)], stm_v)
    pltpu.sync_copy(tbl_hbm, tbl_v)
    pltpu.sync_copy(cst_hbm, cst_v)

    iota16 = lax.iota(jnp.int32, 16)
    bias = [cst_v[pl.ds(c * 16, 16)] for c in range(ACC)]
    wfir = [cst_v[pl.ds((ACC + c) * 16, 16)] for c in range(ACC)]
    wsec = [cst_v[pl.ds((2 * ACC + c) * 16, 16)] for c in range(ACC)]
    l2b = cst_v[pl.ds(3 * ACC * 16, 16)]

    zero = jnp.zeros((16,), jnp.float32)
    tstep = iota16 * TSTRIDE

    def unpacked(word):
        pair = plsc.bitcast(word, jnp.bfloat16)
        return plsc.unpack(pair, format=plsc.PackFormat.INTERLEAVED,
                           preferred_element_type=jnp.float32)

    for h in range(NCHUNK):
        raw_b, raw_w = raws[h % 2][:2]
        pending[0].wait()
        pending[1].wait()
        if h + 1 < NCHUNK:
            pending = issue(h + 1)

        def compact_body(r, carry):
            g = r >> 4
            j = r & 15
            gl = g * L
            for l0 in (0, 16, 32, L - 16):
                widx = (gl + l0) * TSTRIDE + j + tstep
                vb = raw_b.at[r][pl.ds(l0, 16)] * TBL_STRIDE
                vw = raw_w.at[r][pl.ds(l0, 16)] * TBL_STRIDE
                plsc.store_scatter(tb_v, [widx], vb)
                plsc.store_scatter(tw_v, [widx], vw)
            return carry

        lax.fori_loop(0, CHUNK, compact_body, 0)

        def group_body(g, carry):
            gl = g * L

            def l_body(l, accs):
                accb, accw = accs[:ACC], accs[ACC:]
                tbase = (gl + l) * TSTRIDE
                tb = tb_v[pl.ds(tbase, 16)]
                tw = tw_v[pl.ds(tbase, 16)]
                naccb, naccw = [], []
                for p in range(PAIRS):
                    eb, ob = unpacked(plsc.load_gather(tbl_v, [tb + p]))
                    ew, ow = unpacked(plsc.load_gather(tbl_v, [tw + p]))
                    naccb += [accb[2 * p] + eb, accb[2 * p + 1] + ob]
                    naccw += [accw[2 * p] + ew, accw[2 * p + 1] + ow]
                return tuple(naccb) + tuple(naccw)

            accs = lax.fori_loop(0, L, l_body, (zero,) * (2 * ACC))
            accb, accw = accs[:ACC], accs[ACC:]

            out_rows = h * CHUNK + g * 16 + iota16
            stm_g = plsc.load_gather(stm_v, [out_rows])
            m = stm_g == 0
            o = l2b
            for c in range(ACC):
                cb = jnp.where(m, wfir[c], wsec[c])
                cw = jnp.where(m, wsec[c], wfir[c])
                ab = jnp.clip(accb[c] + bias[c], 0.0, 1.0)
                aw = jnp.clip(accw[c] + bias[c], 0.0, 1.0)
                o = o + cb * ab + cw * aw
            plsc.store_scatter(out_v, [out_rows], o)
            return carry

        lax.fori_loop(0, CGROUPS, group_body, 0)

    pltpu.sync_copy(out_v, out_hbm.at[pl.ds(base, ROWS_PER_W)])


@functools.lru_cache(maxsize=1)
def _sc_kernel():
    mesh = plsc.VectorSubcoreMesh(core_axis_name="c", subcore_axis_name="s",
                                  num_cores=NC, num_subcores=NS)
    return pl.kernel(
        _sc_body,
        out_type=jax.ShapeDtypeStruct((BATCH,), jnp.float32),
        mesh=mesh,
        compiler_params=pltpu.CompilerParams(needs_layout_passes=False,
                                             use_tc_tiling_on_sc=True),
        scratch_types=[
            pltpu.VMEM((CHUNK, L), jnp.int32),
            pltpu.VMEM((CHUNK, L), jnp.int32),
            pltpu.VMEM((CHUNK, L), jnp.int32),
            pltpu.VMEM((CHUNK, L), jnp.int32),
            pltpu.VMEM((T_WORDS,), jnp.int32),
            pltpu.VMEM((T_WORDS,), jnp.int32),
            pltpu.VMEM((ROWS_PER_W,), jnp.int32),
            pltpu.VMEM((TBL_WORDS,), jnp.int32),
            pltpu.VMEM((26 * 16,), jnp.float32),
            pltpu.VMEM((ROWS_PER_W,), jnp.float32),
            pltpu.SemaphoreType.DMA,
            pltpu.SemaphoreType.DMA,
            pltpu.SemaphoreType.DMA,
            pltpu.SemaphoreType.DMA,
        ],
    )


def kernel(black_features, white_features, stm, l1_weight, l1_bias,
           l2_weight, l2_bias):
    bf = black_features.astype(jnp.int32)
    wf = white_features.astype(jnp.int32)
    stm32 = stm.astype(jnp.int32)
    packed = lax.bitcast_convert_type(
        l1_weight.astype(jnp.bfloat16).reshape(NUM_FEATURES, PAIRS, 2),
        jnp.int32)
    tbl = jnp.pad(packed, ((0, 0), (0, TBL_STRIDE - PAIRS))).reshape(-1)

    w = l2_weight.reshape(2 * ACC)
    cst = jnp.concatenate([
        jnp.broadcast_to(l1_bias[:, None], (ACC, 16)),
        jnp.broadcast_to(w[:ACC, None], (ACC, 16)),
        jnp.broadcast_to(w[ACC:, None], (ACC, 16)),
        jnp.broadcast_to(l2_bias.reshape(1, 1), (1, 16)),
        jnp.zeros((1, 16), jnp.float32),
    ], axis=0).reshape(-1)

    out = _sc_kernel()(bf, wf, stm32, tbl, cst)
    return out.reshape(BATCH, 1)
